# baseline (speedup 1.0000x reference)
import functools

import jax
import jax.numpy as jnp
import numpy as np
from jax import lax
from jax.experimental import pallas as pl
from jax.experimental.pallas import tpu as pltpu
from jax.experimental.pallas import tpu_sc as plsc

_CLS = 16
_SIZES = [128, 64, 32]
_B = 2
_K = 2000
_NS = 16
_L = 16

_C0 = _SIZES[0] * _SIZES[0] // _NS
_C1 = _SIZES[1] * _SIZES[1] // _NS

_FINE_I = int(np.frombuffer(np.float32(0.02).tobytes(), np.int32)[0])


def _bce(p, t):
    return -(t * jnp.clip(jnp.log(p), -100.0, None)
             + (1.0 - t) * jnp.clip(jnp.log(1.0 - p), -100.0, None))


def _smooth_l1(a, b):
    d = jnp.abs(a - b)
    return jnp.where(d < 1.0, 0.5 * d * d, d - 0.5)


def _dense_body(*refs):
    ins = refs[:24]
    big0, big1, big0u, big1u, smalls = refs[24:]
    (t_cls, t_bbox, t_ang, t_cent, s_cls, s_bbox, s_ang, s_cent) = (
        ins[0:3], ins[3:6], ins[6:9], ins[9:12],
        ins[12:15], ins[15:18], ins[18:21], ins[21:24])

    for img in range(_B):
        ln_total = jnp.float32(0.0)
        neg_num = jnp.float32(0.0)
        mv_sum = jnp.float32(0.0)
        npos2 = jnp.float32(0.0)
        wm2 = jnp.float32(0.0)
        dlp2 = jnp.float32(0.0)
        pb2 = jnp.float32(0.0)
        pc2 = jnp.float32(0.0)

        for lvl in range(3):
            tcs = jax.nn.sigmoid(t_cent[lvl][img, 0])
            tp = jax.nn.sigmoid(t_cls[lvl][img])
            if lvl < 2:
                mv = jnp.max(tp * tcs[None], axis=0)
            else:
                mv = jnp.max(tp, axis=0)

            ssig = jax.nn.sigmoid(s_cls[lvl][img])
            lp = jnp.sum(_bce(ssig, tp) * (tp - ssig) ** 2, axis=0)
            ln = jnp.sum(_bce(ssig, jnp.zeros_like(ssig)) * ssig ** 2, axis=0)

            bb = jnp.sum(_smooth_l1(s_bbox[lvl][img], t_bbox[lvl][img]), axis=0)
            bb = bb + _smooth_l1(s_ang[lvl][img, 0], t_ang[lvl][img, 0])
            ce = _bce(jax.nn.sigmoid(s_cent[lvl][img, 0]), tcs)

            dlp = lp - ln
            pb = mv * bb
            pc = mv * ce

            ln_total = ln_total + jnp.sum(ln)
            mv_sum = mv_sum + jnp.sum(mv)
            neg_num = neg_num + jnp.sum(jnp.where(mv > 0.0, lp, ln))

            if lvl == 0:
                big0[img, 0, :, :] = mv
                big0[img, 1, :, :] = dlp
                big0[img, 2, :, :] = pb
                big0[img, 3, :, :] = pc
                big0u[img, :, :] = lax.bitcast_convert_type(mv, jnp.int32)
            elif lvl == 1:
                big1[img, 0, :, :] = mv
                big1[img, 1, :, :] = dlp
                big1[img, 2, :, :] = pb
                big1[img, 3, :, :] = pc
                big1u[img, :, :] = lax.bitcast_convert_type(mv, jnp.int32)
            else:
                m2 = mv > 0.02
                npos2 = npos2 + jnp.sum(m2.astype(jnp.float32))
                wm2 = wm2 + jnp.sum(jnp.where(m2, mv, 0.0))
                dlp2 = dlp2 + jnp.sum(jnp.where(m2, dlp, 0.0))
                pb2 = pb2 + jnp.sum(jnp.where(m2, pb, 0.0))
                pc2 = pc2 + jnp.sum(jnp.where(m2, pc, 0.0))

        lane = lax.broadcasted_iota(jnp.int32, (1, 128), 1)
        row = jnp.zeros((1, 128), jnp.float32)
        for k, v in enumerate([ln_total, neg_num, mv_sum, npos2, wm2,
                               dlp2, pb2, pc2]):
            row = jnp.where(lane == k, v, row)
        smalls[pl.ds(img, 1), :] = row


def _dense_stage(args):
    return pl.pallas_call(
        _dense_body,
        out_shape=[
            jax.ShapeDtypeStruct((_B, 4, 128, 128), jnp.float32),
            jax.ShapeDtypeStruct((_B, 4, 64, 64), jnp.float32),
            jax.ShapeDtypeStruct((_B, 128, 128), jnp.int32),
            jax.ShapeDtypeStruct((_B, 64, 64), jnp.int32),
            jax.ShapeDtypeStruct((_B, 128), jnp.float32),
        ],
    )(*args)


def _lane_iota():
    return lax.iota(jnp.int32, _L)


def _bcast(vec, lane):
    return vec[jnp.full((_L,), lane, jnp.int32)]


def _lanesum(vec):
    li = _lane_iota()
    for d in (1, 2, 4, 8):
        vec = vec + vec[li ^ d]
    return vec


def _prefix_incl(x):
    li = _lane_iota()
    for d in (1, 2, 4, 8):
        shifted = x[jnp.maximum(li - d, 0)]
        x = x + jnp.where(li >= d, shifted, 0)
    return x


def _count_chunks(b0i, b1i, p0, p1, strict):
    cnt0 = jnp.zeros((_L,), jnp.int32)
    for j in range(_C0 // _L):
        u = b0i[pl.ds(j * _L, _L)]
        m = (u > p0) if strict else (u >= p0)
        cnt0 = cnt0 + jnp.where(m, 1, 0)
    cnt1 = jnp.zeros((_L,), jnp.int32)
    for j in range(_C1 // _L):
        u = b1i[pl.ds(j * _L, _L)]
        m = (u > p1) if strict else (u >= p1)
        cnt1 = cnt1 + jnp.where(m, 1, 0)
    return _lanesum(cnt0), _lanesum(cnt1)


def _select_body(big0, big1, big0u, big1u, smalls, out, buf0, buf1, b0i, b1i,
                 sbuf, pubi, rdi, pubf, rdf, outbuf, shared_i, shared_f, sem):
    c = lax.axis_index("c")
    s = lax.axis_index("s")
    li = _lane_iota()

    cps = []
    for arr in range(4):
        cps.append(pltpu.async_copy(
            big0.at[pl.ds((c * 4 + arr) * 16384 + s * _C0, _C0)],
            buf0.at[arr], sem))
        cps.append(pltpu.async_copy(
            big1.at[pl.ds((c * 4 + arr) * 4096 + s * _C1, _C1)],
            buf1.at[arr], sem))
    cps.append(pltpu.async_copy(
        big0u.at[pl.ds(c * 16384 + s * _C0, _C0)], b0i, sem))
    cps.append(pltpu.async_copy(
        big1u.at[pl.ds(c * 4096 + s * _C1, _C1)], b1i, sem))
    cps.append(pltpu.async_copy(smalls.at[pl.ds(c * 128, 128)], sbuf, sem))
    for cp in cps:
        cp.wait()

    def round_fn(i, lohi):
        lo, hi = lohi
        mid = (lo + hi) >> 1
        c0, c1 = _count_chunks(b0i, b1i, _bcast(mid, 0), _bcast(mid, 1),
                               strict=False)
        pub = jnp.where(li == 0, c0, jnp.where(li == 1, c1, 0))
        pubi[...] = pub
        slot = 4 + lax.rem(i, 2)
        pltpu.sync_copy(pubi, shared_i.at[slot, s])
        plsc.subcore_barrier()
        pltpu.sync_copy(shared_i.at[slot], rdi)
        tot = jnp.zeros((_L,), jnp.int32)
        for r in range(_NS):
            tot = tot + rdi[r, :]
        good = tot >= _K
        return (jnp.where(good, mid, lo), jnp.where(good, hi, mid))

    lo0 = jnp.zeros((_L,), jnp.int32)
    hi0 = jnp.full((_L,), 1 << 30, jnp.int32)
    lo, _ = lax.fori_loop(0, 31, round_fn, (lo0, hi0))
    t0 = _bcast(lo, 0)
    t1 = _bcast(lo, 1)

    g0, g1 = _count_chunks(b0i, b1i, t0, t1, strict=True)
    e0 = jnp.zeros((_L,), jnp.int32)
    for j in range(_C0 // _L):
        u = b0i[pl.ds(j * _L, _L)]
        e0 = e0 + jnp.where(u == t0, 1, 0)
    e1 = jnp.zeros((_L,), jnp.int32)
    for j in range(_C1 // _L):
        u = b1i[pl.ds(j * _L, _L)]
        e1 = e1 + jnp.where(u == t1, 1, 0)
    pub = (jnp.where(li == 0, g0, 0) + jnp.where(li == 1, g1, 0)
           + jnp.where(li == 2, _lanesum(e0), 0)
           + jnp.where(li == 3, _lanesum(e1), 0))
    pubi[...] = pub
    pltpu.sync_copy(pubi, shared_i.at[6, s])
    plsc.subcore_barrier()
    pltpu.sync_copy(shared_i.at[6], rdi)
    s_vec = jnp.zeros((_L,), jnp.int32) + s
    tot = jnp.zeros((_L,), jnp.int32)
    pref = jnp.zeros((_L,), jnp.int32)
    for r in range(_NS):
        row = rdi[r, :]
        tot = tot + row
        pref = pref + row * jnp.minimum(jnp.maximum(s_vec - r, 0), 1)
    b0 = (_K - _bcast(tot, 0)) - _bcast(pref, 2)
    b1 = (_K - _bcast(tot, 1)) - _bcast(pref, 3)

    npos = jnp.zeros((_L,), jnp.int32)
    wm = jnp.zeros((_L,), jnp.float32)
    dlp = jnp.zeros((_L,), jnp.float32)
    pb = jnp.zeros((_L,), jnp.float32)
    pc = jnp.zeros((_L,), jnp.float32)
    for lvl, buf, bi, n, tt, bb in ((0, buf0, b0i, _C0, t0, b0),
                                    (1, buf1, b1i, _C1, t1, b1)):
        carry = jnp.zeros((_L,), jnp.int32)
        for j in range(n // _L):
            sl = pl.ds(j * _L, _L)
            mvv = buf[0, sl]
            u = bi[sl]
            tie_i = jnp.where(u == tt, 1, 0)
            incl = _prefix_incl(tie_i)
            cond_i = jnp.minimum(jnp.maximum(bb - incl - carry + 1, 0), 1)
            gt_i = jnp.where(u > tt, 1, 0)
            sel_i = gt_i + tie_i * cond_i
            carry = carry + _bcast(incl, _L - 1)
            fine_i = jnp.where(u > _FINE_I, 1, 0)
            m_i = sel_i * fine_i
            m_f = m_i.astype(jnp.float32)
            npos = npos + m_i
            wm = wm + mvv * m_f
            dlp = dlp + buf[1, sl] * m_f
            pb = pb + buf[2, sl] * m_f
            pc = pc + buf[3, sl] * m_f

    pubv = (jnp.where(li == 0, _lanesum(npos).astype(jnp.float32), 0.0)
            + jnp.where(li == 1, _lanesum(wm), 0.0)
            + jnp.where(li == 2, _lanesum(dlp), 0.0)
            + jnp.where(li == 3, _lanesum(pb), 0.0)
            + jnp.where(li == 4, _lanesum(pc), 0.0))
    pubf[...] = pubv
    pltpu.sync_copy(pubf, shared_f.at[2, s])
    plsc.subcore_barrier()

    @pl.when(s == 0)
    def _():
        pltpu.sync_copy(shared_f.at[2], rdf)
        acc = jnp.zeros((_L,), jnp.float32)
        for r in range(_NS):
            acc = acc + rdf[r, :]
        sm = sbuf[pl.ds(0, _L)]
        ln_total = _bcast(sm, 0)
        neg_num = _bcast(sm, 1)
        mv_sum = _bcast(sm, 2)
        npos_f = _bcast(acc, 0) + _bcast(sm, 3)
        wm_sum = _bcast(acc, 1) + _bcast(sm, 4)
        dlp_sum = _bcast(acc, 2) + _bcast(sm, 5)
        pb_sum = _bcast(acc, 3) + _bcast(sm, 6)
        pc_sum = _bcast(acc, 4) + _bcast(sm, 7)

        hp = jnp.minimum(npos_f, 1.0)
        pos_num = ln_total + dlp_sum
        wm_safe = wm_sum * hp + (1.0 - hp)
        loss_cls = hp * (pos_num / wm_safe) + (1.0 - hp) * (neg_num / mv_sum)
        npf = jnp.maximum(npos_f, 1.0)
        loss_bbox = hp * (pb_sum / (npf * 5.0) * 10.0)
        loss_cent = hp * (pc_sum / npf * 10.0)

        ov = (jnp.where(li == 0, loss_cls, 0.0)
              + jnp.where(li == 1, loss_bbox, 0.0)
              + jnp.where(li == 2, loss_cent, 0.0))
        outbuf[...] = ov
        pltpu.sync_copy(outbuf, out.at[pl.ds(c * _L, _L)])


_select_stage = functools.partial(
    pl.kernel,
    _select_body,
    out_type=jax.ShapeDtypeStruct((2 * _L,), jnp.float32),
    mesh=plsc.VectorSubcoreMesh(core_axis_name="c", subcore_axis_name="s",
                                num_cores=2, num_subcores=_NS),
    scratch_types=[
        pltpu.VMEM((4, _C0), jnp.float32),
        pltpu.VMEM((4, _C1), jnp.float32),
        pltpu.VMEM((_C0,), jnp.int32),
        pltpu.VMEM((_C1,), jnp.int32),
        pltpu.VMEM((128,), jnp.float32),
        pltpu.VMEM((_L,), jnp.int32),
        pltpu.VMEM((_NS, _L), jnp.int32),
        pltpu.VMEM((_L,), jnp.float32),
        pltpu.VMEM((_NS, _L), jnp.float32),
        pltpu.VMEM((_L,), jnp.float32),
        pltpu.VMEM_SHARED((8, _NS, _L), jnp.int32),
        pltpu.VMEM_SHARED((4, _NS, _L), jnp.float32),
        pltpu.SemaphoreType.DMA,
    ],
)


def kernel(t_cls_0, t_cls_1, t_cls_2, t_bbox_0, t_bbox_1, t_bbox_2,
           t_ang_0, t_ang_1, t_ang_2, t_cent_0, t_cent_1, t_cent_2,
           s_cls_0, s_cls_1, s_cls_2, s_bbox_0, s_bbox_1, s_bbox_2,
           s_ang_0, s_ang_1, s_ang_2, s_cent_0, s_cent_1, s_cent_2):
    args = (t_cls_0, t_cls_1, t_cls_2, t_bbox_0, t_bbox_1, t_bbox_2,
            t_ang_0, t_ang_1, t_ang_2, t_cent_0, t_cent_1, t_cent_2,
            s_cls_0, s_cls_1, s_cls_2, s_bbox_0, s_bbox_1, s_bbox_2,
            s_ang_0, s_ang_1, s_ang_2, s_cent_0, s_cent_1, s_cent_2)
    big0, big1, big0u, big1u, smalls = _dense_stage(args)
    out = _select_stage()(big0.reshape(-1), big1.reshape(-1),
                          big0u.reshape(-1), big1u.reshape(-1),
                          smalls.reshape(-1))
    loss_cls = (out[0] + out[_L]) * 0.5
    loss_bbox = (out[1] + out[_L + 1]) * 0.5
    loss_cent = (out[2] + out[_L + 2]) * 0.5
    return (loss_cls, loss_bbox, loss_cent)


# speedup vs baseline: 1.0708x; 1.0708x over previous
import functools

import jax
import jax.numpy as jnp
import numpy as np
from jax import lax
from jax.experimental import pallas as pl
from jax.experimental.pallas import tpu as pltpu
from jax.experimental.pallas import tpu_sc as plsc

_CLS = 16
_SIZES = [128, 64, 32]
_B = 2
_K = 2000
_NS = 16
_L = 16

_C0 = _SIZES[0] * _SIZES[0] // _NS
_C1 = _SIZES[1] * _SIZES[1] // _NS

_FINE_I = int(np.frombuffer(np.float32(0.02).tobytes(), np.int32)[0])


def _bce(p, t):
    return -(t * jnp.clip(jnp.log(p), -100.0, None)
             + (1.0 - t) * jnp.clip(jnp.log(1.0 - p), -100.0, None))


def _smooth_l1(a, b):
    d = jnp.abs(a - b)
    return jnp.where(d < 1.0, 0.5 * d * d, d - 0.5)


def _dense_body(*refs):
    ins = refs[:24]
    big0, big1, big0u, big1u, smalls = refs[24:]
    (t_cls, t_bbox, t_ang, t_cent, s_cls, s_bbox, s_ang, s_cent) = (
        ins[0:3], ins[3:6], ins[6:9], ins[9:12],
        ins[12:15], ins[15:18], ins[18:21], ins[21:24])

    for img in range(_B):
        ln_total = jnp.float32(0.0)
        neg_num = jnp.float32(0.0)
        mv_sum = jnp.float32(0.0)
        npos2 = jnp.float32(0.0)
        wm2 = jnp.float32(0.0)
        dlp2 = jnp.float32(0.0)
        pb2 = jnp.float32(0.0)
        pc2 = jnp.float32(0.0)

        for lvl in range(3):
            tcs = jax.nn.sigmoid(t_cent[lvl][img, 0])
            tp = jax.nn.sigmoid(t_cls[lvl][img])
            if lvl < 2:
                mv = jnp.max(tp * tcs[None], axis=0)
            else:
                mv = jnp.max(tp, axis=0)

            ssig = jax.nn.sigmoid(s_cls[lvl][img])
            lp = jnp.sum(_bce(ssig, tp) * (tp - ssig) ** 2, axis=0)
            ln = jnp.sum(_bce(ssig, jnp.zeros_like(ssig)) * ssig ** 2, axis=0)

            bb = jnp.sum(_smooth_l1(s_bbox[lvl][img], t_bbox[lvl][img]), axis=0)
            bb = bb + _smooth_l1(s_ang[lvl][img, 0], t_ang[lvl][img, 0])
            ce = _bce(jax.nn.sigmoid(s_cent[lvl][img, 0]), tcs)

            dlp = lp - ln
            pb = mv * bb
            pc = mv * ce

            ln_total = ln_total + jnp.sum(ln)
            mv_sum = mv_sum + jnp.sum(mv)
            neg_num = neg_num + jnp.sum(jnp.where(mv > 0.0, lp, ln))

            if lvl == 0:
                big0[img, 0, :, :] = mv
                big0[img, 1, :, :] = dlp
                big0[img, 2, :, :] = pb
                big0[img, 3, :, :] = pc
                big0u[img, :, :] = lax.bitcast_convert_type(mv, jnp.int32)
            elif lvl == 1:
                big1[img, 0, :, :] = mv
                big1[img, 1, :, :] = dlp
                big1[img, 2, :, :] = pb
                big1[img, 3, :, :] = pc
                big1u[img, :, :] = lax.bitcast_convert_type(mv, jnp.int32)
            else:
                m2 = mv > 0.02
                npos2 = npos2 + jnp.sum(m2.astype(jnp.float32))
                wm2 = wm2 + jnp.sum(jnp.where(m2, mv, 0.0))
                dlp2 = dlp2 + jnp.sum(jnp.where(m2, dlp, 0.0))
                pb2 = pb2 + jnp.sum(jnp.where(m2, pb, 0.0))
                pc2 = pc2 + jnp.sum(jnp.where(m2, pc, 0.0))

        lane = lax.broadcasted_iota(jnp.int32, (1, 128), 1)
        row = jnp.zeros((1, 128), jnp.float32)
        for k, v in enumerate([ln_total, neg_num, mv_sum, npos2, wm2,
                               dlp2, pb2, pc2]):
            row = jnp.where(lane == k, v, row)
        smalls[pl.ds(img, 1), :] = row


def _dense_stage(args):
    return pl.pallas_call(
        _dense_body,
        out_shape=[
            jax.ShapeDtypeStruct((_B, 4, 128, 128), jnp.float32),
            jax.ShapeDtypeStruct((_B, 4, 64, 64), jnp.float32),
            jax.ShapeDtypeStruct((_B, 128, 128), jnp.int32),
            jax.ShapeDtypeStruct((_B, 64, 64), jnp.int32),
            jax.ShapeDtypeStruct((_B, 128), jnp.float32),
        ],
    )(*args)


def _lane_iota():
    return lax.iota(jnp.int32, _L)


def _bcast(vec, lane):
    return vec[jnp.full((_L,), lane, jnp.int32)]


def _lanesum(vec):
    li = _lane_iota()
    for d in (1, 2, 4, 8):
        vec = vec + vec[li ^ d]
    return vec


def _prefix_incl(x):
    li = _lane_iota()
    for d in (1, 2, 4, 8):
        shifted = x[jnp.maximum(li - d, 0)]
        x = x + jnp.where(li >= d, shifted, 0)
    return x


def _count_chunks(b0i, b1i, p0, p1, strict):
    cnt0 = jnp.zeros((_L,), jnp.int32)
    for j in range(_C0 // _L):
        u = b0i[pl.ds(j * _L, _L)]
        m = (u > p0) if strict else (u >= p0)
        cnt0 = cnt0 + jnp.where(m, 1, 0)
    cnt1 = jnp.zeros((_L,), jnp.int32)
    for j in range(_C1 // _L):
        u = b1i[pl.ds(j * _L, _L)]
        m = (u > p1) if strict else (u >= p1)
        cnt1 = cnt1 + jnp.where(m, 1, 0)
    return _lanesum(cnt0), _lanesum(cnt1)


def _count3_chunks(b0i, b1i, ps0, ps1):
    c0 = [jnp.zeros((_L,), jnp.int32) for _ in range(3)]
    for j in range(_C0 // _L):
        u = b0i[pl.ds(j * _L, _L)]
        for k in range(3):
            c0[k] = c0[k] + jnp.where(u >= ps0[k], 1, 0)
    c1 = [jnp.zeros((_L,), jnp.int32) for _ in range(3)]
    for j in range(_C1 // _L):
        u = b1i[pl.ds(j * _L, _L)]
        for k in range(3):
            c1[k] = c1[k] + jnp.where(u >= ps1[k], 1, 0)
    return [_lanesum(v) for v in c0], [_lanesum(v) for v in c1]


def _select_body(big0, big1, big0u, big1u, smalls, out, buf0, buf1, b0i, b1i,
                 sbuf, pubi, rdi, pubf, rdf, outbuf, shared_i, shared_f, sem):
    c = lax.axis_index("c")
    s = lax.axis_index("s")
    li = _lane_iota()

    cps = []
    for arr in range(4):
        cps.append(pltpu.async_copy(
            big0.at[pl.ds((c * 4 + arr) * 16384 + s * _C0, _C0)],
            buf0.at[arr], sem))
        cps.append(pltpu.async_copy(
            big1.at[pl.ds((c * 4 + arr) * 4096 + s * _C1, _C1)],
            buf1.at[arr], sem))
    cps.append(pltpu.async_copy(
        big0u.at[pl.ds(c * 16384 + s * _C0, _C0)], b0i, sem))
    cps.append(pltpu.async_copy(
        big1u.at[pl.ds(c * 4096 + s * _C1, _C1)], b1i, sem))
    cps.append(pltpu.async_copy(smalls.at[pl.ds(c * 128, 128)], sbuf, sem))
    for cp in cps:
        cp.wait()

    def round_fn(i, lohi):
        lo, hi = lohi
        d = (hi - lo) >> 2
        p1 = lo + d
        p2 = lo + d + d
        p3 = lo + d + d + d
        c0s, c1s = _count3_chunks(
            b0i, b1i,
            [_bcast(p1, 0), _bcast(p2, 0), _bcast(p3, 0)],
            [_bcast(p1, 1), _bcast(p2, 1), _bcast(p3, 1)])
        pub = jnp.zeros((_L,), jnp.int32)
        for k in range(3):
            pub = (pub + jnp.where(li == k, c0s[k], 0)
                   + jnp.where(li == 3 + k, c1s[k], 0))
        pubi[...] = pub
        slot = 4 + lax.rem(i, 2)
        pltpu.sync_copy(pubi, shared_i.at[slot, s])
        plsc.subcore_barrier()
        pltpu.sync_copy(shared_i.at[slot], rdi)
        tot = jnp.zeros((_L,), jnp.int32)
        for r in range(_NS):
            tot = tot + rdi[r, :]
        sel = jnp.minimum(li, 1) * 3
        g1 = tot[sel] >= _K
        g2 = tot[sel + 1] >= _K
        g3 = tot[sel + 2] >= _K
        lo2 = jnp.where(g3, p3, jnp.where(g2, p2, jnp.where(g1, p1, lo)))
        hi2 = jnp.where(g3, hi, jnp.where(g2, p3, jnp.where(g1, p2, p1)))
        return (lo2, hi2)

    lo0 = jnp.zeros((_L,), jnp.int32)
    hi0 = jnp.full((_L,), 1 << 30, jnp.int32)
    lo, _ = lax.fori_loop(0, 15, round_fn, (lo0, hi0))
    t0 = _bcast(lo, 0)
    t1 = _bcast(lo, 1)

    g0, g1 = _count_chunks(b0i, b1i, t0, t1, strict=True)
    e0 = jnp.zeros((_L,), jnp.int32)
    for j in range(_C0 // _L):
        u = b0i[pl.ds(j * _L, _L)]
        e0 = e0 + jnp.where(u == t0, 1, 0)
    e1 = jnp.zeros((_L,), jnp.int32)
    for j in range(_C1 // _L):
        u = b1i[pl.ds(j * _L, _L)]
        e1 = e1 + jnp.where(u == t1, 1, 0)
    pub = (jnp.where(li == 0, g0, 0) + jnp.where(li == 1, g1, 0)
           + jnp.where(li == 2, _lanesum(e0), 0)
           + jnp.where(li == 3, _lanesum(e1), 0))
    pubi[...] = pub
    pltpu.sync_copy(pubi, shared_i.at[6, s])
    plsc.subcore_barrier()
    pltpu.sync_copy(shared_i.at[6], rdi)
    s_vec = jnp.zeros((_L,), jnp.int32) + s
    tot = jnp.zeros((_L,), jnp.int32)
    pref = jnp.zeros((_L,), jnp.int32)
    for r in range(_NS):
        row = rdi[r, :]
        tot = tot + row
        pref = pref + row * jnp.minimum(jnp.maximum(s_vec - r, 0), 1)
    b0 = (_K - _bcast(tot, 0)) - _bcast(pref, 2)
    b1 = (_K - _bcast(tot, 1)) - _bcast(pref, 3)

    npos = jnp.zeros((_L,), jnp.int32)
    wm = jnp.zeros((_L,), jnp.float32)
    dlp = jnp.zeros((_L,), jnp.float32)
    pb = jnp.zeros((_L,), jnp.float32)
    pc = jnp.zeros((_L,), jnp.float32)
    for lvl, buf, bi, n, tt, bb in ((0, buf0, b0i, _C0, t0, b0),
                                    (1, buf1, b1i, _C1, t1, b1)):
        carry = jnp.zeros((_L,), jnp.int32)
        for j in range(n // _L):
            sl = pl.ds(j * _L, _L)
            mvv = buf[0, sl]
            u = bi[sl]
            tie_i = jnp.where(u == tt, 1, 0)
            incl = _prefix_incl(tie_i)
            cond_i = jnp.minimum(jnp.maximum(bb - incl - carry + 1, 0), 1)
            gt_i = jnp.where(u > tt, 1, 0)
            sel_i = gt_i + tie_i * cond_i
            carry = carry + _bcast(incl, _L - 1)
            fine_i = jnp.where(u > _FINE_I, 1, 0)
            m_i = sel_i * fine_i
            m_f = m_i.astype(jnp.float32)
            npos = npos + m_i
            wm = wm + mvv * m_f
            dlp = dlp + buf[1, sl] * m_f
            pb = pb + buf[2, sl] * m_f
            pc = pc + buf[3, sl] * m_f

    pubv = (jnp.where(li == 0, _lanesum(npos).astype(jnp.float32), 0.0)
            + jnp.where(li == 1, _lanesum(wm), 0.0)
            + jnp.where(li == 2, _lanesum(dlp), 0.0)
            + jnp.where(li == 3, _lanesum(pb), 0.0)
            + jnp.where(li == 4, _lanesum(pc), 0.0))
    pubf[...] = pubv
    pltpu.sync_copy(pubf, shared_f.at[2, s])
    plsc.subcore_barrier()

    @pl.when(s == 0)
    def _():
        pltpu.sync_copy(shared_f.at[2], rdf)
        acc = jnp.zeros((_L,), jnp.float32)
        for r in range(_NS):
            acc = acc + rdf[r, :]
        sm = sbuf[pl.ds(0, _L)]
        ln_total = _bcast(sm, 0)
        neg_num = _bcast(sm, 1)
        mv_sum = _bcast(sm, 2)
        npos_f = _bcast(acc, 0) + _bcast(sm, 3)
        wm_sum = _bcast(acc, 1) + _bcast(sm, 4)
        dlp_sum = _bcast(acc, 2) + _bcast(sm, 5)
        pb_sum = _bcast(acc, 3) + _bcast(sm, 6)
        pc_sum = _bcast(acc, 4) + _bcast(sm, 7)

        hp = jnp.minimum(npos_f, 1.0)
        pos_num = ln_total + dlp_sum
        wm_safe = wm_sum * hp + (1.0 - hp)
        loss_cls = hp * (pos_num / wm_safe) + (1.0 - hp) * (neg_num / mv_sum)
        npf = jnp.maximum(npos_f, 1.0)
        loss_bbox = hp * (pb_sum / (npf * 5.0) * 10.0)
        loss_cent = hp * (pc_sum / npf * 10.0)

        ov = (jnp.where(li == 0, loss_cls, 0.0)
              + jnp.where(li == 1, loss_bbox, 0.0)
              + jnp.where(li == 2, loss_cent, 0.0))
        outbuf[...] = ov
        pltpu.sync_copy(outbuf, out.at[pl.ds(c * _L, _L)])


_select_stage = functools.partial(
    pl.kernel,
    _select_body,
    out_type=jax.ShapeDtypeStruct((2 * _L,), jnp.float32),
    mesh=plsc.VectorSubcoreMesh(core_axis_name="c", subcore_axis_name="s",
                                num_cores=2, num_subcores=_NS),
    scratch_types=[
        pltpu.VMEM((4, _C0), jnp.float32),
        pltpu.VMEM((4, _C1), jnp.float32),
        pltpu.VMEM((_C0,), jnp.int32),
        pltpu.VMEM((_C1,), jnp.int32),
        pltpu.VMEM((128,), jnp.float32),
        pltpu.VMEM((_L,), jnp.int32),
        pltpu.VMEM((_NS, _L), jnp.int32),
        pltpu.VMEM((_L,), jnp.float32),
        pltpu.VMEM((_NS, _L), jnp.float32),
        pltpu.VMEM((_L,), jnp.float32),
        pltpu.VMEM_SHARED((8, _NS, _L), jnp.int32),
        pltpu.VMEM_SHARED((4, _NS, _L), jnp.float32),
        pltpu.SemaphoreType.DMA,
    ],
)


def kernel(t_cls_0, t_cls_1, t_cls_2, t_bbox_0, t_bbox_1, t_bbox_2,
           t_ang_0, t_ang_1, t_ang_2, t_cent_0, t_cent_1, t_cent_2,
           s_cls_0, s_cls_1, s_cls_2, s_bbox_0, s_bbox_1, s_bbox_2,
           s_ang_0, s_ang_1, s_ang_2, s_cent_0, s_cent_1, s_cent_2):
    args = (t_cls_0, t_cls_1, t_cls_2, t_bbox_0, t_bbox_1, t_bbox_2,
            t_ang_0, t_ang_1, t_ang_2, t_cent_0, t_cent_1, t_cent_2,
            s_cls_0, s_cls_1, s_cls_2, s_bbox_0, s_bbox_1, s_bbox_2,
            s_ang_0, s_ang_1, s_ang_2, s_cent_0, s_cent_1, s_cent_2)
    big0, big1, big0u, big1u, smalls = _dense_stage(args)
    out = _select_stage()(big0.reshape(-1), big1.reshape(-1),
                          big0u.reshape(-1), big1u.reshape(-1),
                          smalls.reshape(-1))
    loss_cls = (out[0] + out[_L]) * 0.5
    loss_bbox = (out[1] + out[_L + 1]) * 0.5
    loss_cent = (out[2] + out[_L + 2]) * 0.5
    return (loss_cls, loss_bbox, loss_cent)


# speedup vs baseline: 1.0816x; 1.0100x over previous
import functools

import jax
import jax.numpy as jnp
import numpy as np
from jax import lax
from jax.experimental import pallas as pl
from jax.experimental.pallas import tpu as pltpu
from jax.experimental.pallas import tpu_sc as plsc

_CLS = 16
_SIZES = [128, 64, 32]
_B = 2
_K = 2000
_NS = 16
_L = 16

_C0 = _SIZES[0] * _SIZES[0] // _NS
_C1 = _SIZES[1] * _SIZES[1] // _NS

_FINE_I = int(np.frombuffer(np.float32(0.02).tobytes(), np.int32)[0])


def _bce(p, t):
    return -(t * jnp.clip(jnp.log(p), -100.0, None)
             + (1.0 - t) * jnp.clip(jnp.log(1.0 - p), -100.0, None))


def _smooth_l1(a, b):
    d = jnp.abs(a - b)
    return jnp.where(d < 1.0, 0.5 * d * d, d - 0.5)


def _dense_body(*refs):
    ins = refs[:24]
    big0, big1, big0u, big1u, smalls = refs[24:]
    (t_cls, t_bbox, t_ang, t_cent, s_cls, s_bbox, s_ang, s_cent) = (
        ins[0:3], ins[3:6], ins[6:9], ins[9:12],
        ins[12:15], ins[15:18], ins[18:21], ins[21:24])

    for img in range(_B):
        ln_total = jnp.float32(0.0)
        neg_num = jnp.float32(0.0)
        mv_sum = jnp.float32(0.0)
        npos2 = jnp.float32(0.0)
        wm2 = jnp.float32(0.0)
        dlp2 = jnp.float32(0.0)
        pb2 = jnp.float32(0.0)
        pc2 = jnp.float32(0.0)

        for lvl in range(3):
            tcs = jax.nn.sigmoid(t_cent[lvl][img, 0])
            tp = jax.nn.sigmoid(t_cls[lvl][img])
            if lvl < 2:
                mv = jnp.max(tp * tcs[None], axis=0)
            else:
                mv = jnp.max(tp, axis=0)

            sx = s_cls[lvl][img]
            z = jnp.exp(-sx)
            ssig = 1.0 / (1.0 + z)
            sp = jnp.log(1.0 + z)
            logp = jnp.clip(-sp, -100.0, None)
            log1mp = jnp.clip(-sx - sp, -100.0, None)
            lp = jnp.sum(-(tp * logp + (1.0 - tp) * log1mp)
                         * (tp - ssig) ** 2, axis=0)
            ln = jnp.sum(-log1mp * ssig ** 2, axis=0)

            bb = jnp.sum(_smooth_l1(s_bbox[lvl][img], t_bbox[lvl][img]), axis=0)
            bb = bb + _smooth_l1(s_ang[lvl][img, 0], t_ang[lvl][img, 0])
            cx = s_cent[lvl][img, 0]
            z2 = jnp.exp(-cx)
            sp2 = jnp.log(1.0 + z2)
            ce = -(tcs * jnp.clip(-sp2, -100.0, None)
                   + (1.0 - tcs) * jnp.clip(-cx - sp2, -100.0, None))

            dlp = lp - ln
            pb = mv * bb
            pc = mv * ce

            ln_total = ln_total + jnp.sum(ln)
            mv_sum = mv_sum + jnp.sum(mv)
            neg_num = neg_num + jnp.sum(jnp.where(mv > 0.0, lp, ln))

            if lvl == 0:
                big0[img, 0, :, :] = mv
                big0[img, 1, :, :] = dlp
                big0[img, 2, :, :] = pb
                big0[img, 3, :, :] = pc
                big0u[img, :, :] = lax.bitcast_convert_type(mv, jnp.int32)
            elif lvl == 1:
                big1[img, 0, :, :] = mv
                big1[img, 1, :, :] = dlp
                big1[img, 2, :, :] = pb
                big1[img, 3, :, :] = pc
                big1u[img, :, :] = lax.bitcast_convert_type(mv, jnp.int32)
            else:
                m2 = mv > 0.02
                npos2 = npos2 + jnp.sum(m2.astype(jnp.float32))
                wm2 = wm2 + jnp.sum(jnp.where(m2, mv, 0.0))
                dlp2 = dlp2 + jnp.sum(jnp.where(m2, dlp, 0.0))
                pb2 = pb2 + jnp.sum(jnp.where(m2, pb, 0.0))
                pc2 = pc2 + jnp.sum(jnp.where(m2, pc, 0.0))

        lane = lax.broadcasted_iota(jnp.int32, (1, 128), 1)
        row = jnp.zeros((1, 128), jnp.float32)
        for k, v in enumerate([ln_total, neg_num, mv_sum, npos2, wm2,
                               dlp2, pb2, pc2]):
            row = jnp.where(lane == k, v, row)
        smalls[pl.ds(img, 1), :] = row


def _dense_stage(args):
    return pl.pallas_call(
        _dense_body,
        out_shape=[
            jax.ShapeDtypeStruct((_B, 4, 128, 128), jnp.float32),
            jax.ShapeDtypeStruct((_B, 4, 64, 64), jnp.float32),
            jax.ShapeDtypeStruct((_B, 128, 128), jnp.int32),
            jax.ShapeDtypeStruct((_B, 64, 64), jnp.int32),
            jax.ShapeDtypeStruct((_B, 128), jnp.float32),
        ],
    )(*args)


def _lane_iota():
    return lax.iota(jnp.int32, _L)


def _bcast(vec, lane):
    return vec[jnp.full((_L,), lane, jnp.int32)]


def _lanesum(vec):
    li = _lane_iota()
    for d in (1, 2, 4, 8):
        vec = vec + vec[li ^ d]
    return vec


def _prefix_incl(x):
    li = _lane_iota()
    for d in (1, 2, 4, 8):
        shifted = x[jnp.maximum(li - d, 0)]
        x = x + jnp.where(li >= d, shifted, 0)
    return x


def _count_chunks(b0i, b1i, p0, p1, strict):
    cnt0 = jnp.zeros((_L,), jnp.int32)
    for j in range(_C0 // _L):
        u = b0i[pl.ds(j * _L, _L)]
        m = (u > p0) if strict else (u >= p0)
        cnt0 = cnt0 + jnp.where(m, 1, 0)
    cnt1 = jnp.zeros((_L,), jnp.int32)
    for j in range(_C1 // _L):
        u = b1i[pl.ds(j * _L, _L)]
        m = (u > p1) if strict else (u >= p1)
        cnt1 = cnt1 + jnp.where(m, 1, 0)
    return _lanesum(cnt0), _lanesum(cnt1)


def _count3_chunks(b0i, b1i, ps0, ps1):
    c0 = [jnp.zeros((_L,), jnp.int32) for _ in range(3)]
    for j in range(_C0 // _L):
        u = b0i[pl.ds(j * _L, _L)]
        for k in range(3):
            c0[k] = c0[k] + jnp.where(u >= ps0[k], 1, 0)
    c1 = [jnp.zeros((_L,), jnp.int32) for _ in range(3)]
    for j in range(_C1 // _L):
        u = b1i[pl.ds(j * _L, _L)]
        for k in range(3):
            c1[k] = c1[k] + jnp.where(u >= ps1[k], 1, 0)
    return [_lanesum(v) for v in c0], [_lanesum(v) for v in c1]


def _select_body(big0, big1, big0u, big1u, smalls, out, buf0, buf1, b0i, b1i,
                 sbuf, pubi, rdi, pubf, rdf, outbuf, shared_i, shared_f, sem):
    c = lax.axis_index("c")
    s = lax.axis_index("s")
    li = _lane_iota()

    cps = []
    for arr in range(4):
        cps.append(pltpu.async_copy(
            big0.at[pl.ds((c * 4 + arr) * 16384 + s * _C0, _C0)],
            buf0.at[arr], sem))
        cps.append(pltpu.async_copy(
            big1.at[pl.ds((c * 4 + arr) * 4096 + s * _C1, _C1)],
            buf1.at[arr], sem))
    cps.append(pltpu.async_copy(
        big0u.at[pl.ds(c * 16384 + s * _C0, _C0)], b0i, sem))
    cps.append(pltpu.async_copy(
        big1u.at[pl.ds(c * 4096 + s * _C1, _C1)], b1i, sem))
    cps.append(pltpu.async_copy(smalls.at[pl.ds(c * 128, 128)], sbuf, sem))
    for cp in cps:
        cp.wait()

    def round_fn(i, lohi):
        lo, hi = lohi
        d = (hi - lo) >> 2
        p1 = lo + d
        p2 = lo + d + d
        p3 = lo + d + d + d
        c0s, c1s = _count3_chunks(
            b0i, b1i,
            [_bcast(p1, 0), _bcast(p2, 0), _bcast(p3, 0)],
            [_bcast(p1, 1), _bcast(p2, 1), _bcast(p3, 1)])
        pub = jnp.zeros((_L,), jnp.int32)
        for k in range(3):
            pub = (pub + jnp.where(li == k, c0s[k], 0)
                   + jnp.where(li == 3 + k, c1s[k], 0))
        pubi[...] = pub
        slot = 4 + lax.rem(i, 2)
        pltpu.sync_copy(pubi, shared_i.at[slot, s])
        plsc.subcore_barrier()
        pltpu.sync_copy(shared_i.at[slot], rdi)
        tot = jnp.zeros((_L,), jnp.int32)
        for r in range(_NS):
            tot = tot + rdi[r, :]
        sel = jnp.minimum(li, 1) * 3
        g1 = tot[sel] >= _K
        g2 = tot[sel + 1] >= _K
        g3 = tot[sel + 2] >= _K
        lo2 = jnp.where(g3, p3, jnp.where(g2, p2, jnp.where(g1, p1, lo)))
        hi2 = jnp.where(g3, hi, jnp.where(g2, p3, jnp.where(g1, p2, p1)))
        return (lo2, hi2)

    lo0 = jnp.zeros((_L,), jnp.int32)
    hi0 = jnp.full((_L,), 1 << 30, jnp.int32)
    lo, _ = lax.fori_loop(0, 15, round_fn, (lo0, hi0))
    t0 = _bcast(lo, 0)
    t1 = _bcast(lo, 1)

    g0, g1 = _count_chunks(b0i, b1i, t0, t1, strict=True)
    e0 = jnp.zeros((_L,), jnp.int32)
    for j in range(_C0 // _L):
        u = b0i[pl.ds(j * _L, _L)]
        e0 = e0 + jnp.where(u == t0, 1, 0)
    e1 = jnp.zeros((_L,), jnp.int32)
    for j in range(_C1 // _L):
        u = b1i[pl.ds(j * _L, _L)]
        e1 = e1 + jnp.where(u == t1, 1, 0)
    pub = (jnp.where(li == 0, g0, 0) + jnp.where(li == 1, g1, 0)
           + jnp.where(li == 2, _lanesum(e0), 0)
           + jnp.where(li == 3, _lanesum(e1), 0))
    pubi[...] = pub
    pltpu.sync_copy(pubi, shared_i.at[6, s])
    plsc.subcore_barrier()
    pltpu.sync_copy(shared_i.at[6], rdi)
    s_vec = jnp.zeros((_L,), jnp.int32) + s
    tot = jnp.zeros((_L,), jnp.int32)
    pref = jnp.zeros((_L,), jnp.int32)
    for r in range(_NS):
        row = rdi[r, :]
        tot = tot + row
        pref = pref + row * jnp.minimum(jnp.maximum(s_vec - r, 0), 1)
    b0 = (_K - _bcast(tot, 0)) - _bcast(pref, 2)
    b1 = (_K - _bcast(tot, 1)) - _bcast(pref, 3)

    npos = jnp.zeros((_L,), jnp.int32)
    wm = jnp.zeros((_L,), jnp.float32)
    dlp = jnp.zeros((_L,), jnp.float32)
    pb = jnp.zeros((_L,), jnp.float32)
    pc = jnp.zeros((_L,), jnp.float32)
    for lvl, buf, bi, n, tt, bb in ((0, buf0, b0i, _C0, t0, b0),
                                    (1, buf1, b1i, _C1, t1, b1)):
        carry = jnp.zeros((_L,), jnp.int32)
        for j in range(n // _L):
            sl = pl.ds(j * _L, _L)
            mvv = buf[0, sl]
            u = bi[sl]
            tie_i = jnp.where(u == tt, 1, 0)
            incl = _prefix_incl(tie_i)
            cond_i = jnp.minimum(jnp.maximum(bb - incl - carry + 1, 0), 1)
            gt_i = jnp.where(u > tt, 1, 0)
            sel_i = gt_i + tie_i * cond_i
            carry = carry + _bcast(incl, _L - 1)
            fine_i = jnp.where(u > _FINE_I, 1, 0)
            m_i = sel_i * fine_i
            m_f = m_i.astype(jnp.float32)
            npos = npos + m_i
            wm = wm + mvv * m_f
            dlp = dlp + buf[1, sl] * m_f
            pb = pb + buf[2, sl] * m_f
            pc = pc + buf[3, sl] * m_f

    pubv = (jnp.where(li == 0, _lanesum(npos).astype(jnp.float32), 0.0)
            + jnp.where(li == 1, _lanesum(wm), 0.0)
            + jnp.where(li == 2, _lanesum(dlp), 0.0)
            + jnp.where(li == 3, _lanesum(pb), 0.0)
            + jnp.where(li == 4, _lanesum(pc), 0.0))
    pubf[...] = pubv
    pltpu.sync_copy(pubf, shared_f.at[2, s])
    plsc.subcore_barrier()

    @pl.when(s == 0)
    def _():
        pltpu.sync_copy(shared_f.at[2], rdf)
        acc = jnp.zeros((_L,), jnp.float32)
        for r in range(_NS):
            acc = acc + rdf[r, :]
        sm = sbuf[pl.ds(0, _L)]
        ln_total = _bcast(sm, 0)
        neg_num = _bcast(sm, 1)
        mv_sum = _bcast(sm, 2)
        npos_f = _bcast(acc, 0) + _bcast(sm, 3)
        wm_sum = _bcast(acc, 1) + _bcast(sm, 4)
        dlp_sum = _bcast(acc, 2) + _bcast(sm, 5)
        pb_sum = _bcast(acc, 3) + _bcast(sm, 6)
        pc_sum = _bcast(acc, 4) + _bcast(sm, 7)

        hp = jnp.minimum(npos_f, 1.0)
        pos_num = ln_total + dlp_sum
        wm_safe = wm_sum * hp + (1.0 - hp)
        loss_cls = hp * (pos_num / wm_safe) + (1.0 - hp) * (neg_num / mv_sum)
        npf = jnp.maximum(npos_f, 1.0)
        loss_bbox = hp * (pb_sum / (npf * 5.0) * 10.0)
        loss_cent = hp * (pc_sum / npf * 10.0)

        ov = (jnp.where(li == 0, loss_cls, 0.0)
              + jnp.where(li == 1, loss_bbox, 0.0)
              + jnp.where(li == 2, loss_cent, 0.0))
        outbuf[...] = ov
        pltpu.sync_copy(outbuf, out.at[pl.ds(c * _L, _L)])


_select_stage = functools.partial(
    pl.kernel,
    _select_body,
    out_type=jax.ShapeDtypeStruct((2 * _L,), jnp.float32),
    mesh=plsc.VectorSubcoreMesh(core_axis_name="c", subcore_axis_name="s",
                                num_cores=2, num_subcores=_NS),
    scratch_types=[
        pltpu.VMEM((4, _C0), jnp.float32),
        pltpu.VMEM((4, _C1), jnp.float32),
        pltpu.VMEM((_C0,), jnp.int32),
        pltpu.VMEM((_C1,), jnp.int32),
        pltpu.VMEM((128,), jnp.float32),
        pltpu.VMEM((_L,), jnp.int32),
        pltpu.VMEM((_NS, _L), jnp.int32),
        pltpu.VMEM((_L,), jnp.float32),
        pltpu.VMEM((_NS, _L), jnp.float32),
        pltpu.VMEM((_L,), jnp.float32),
        pltpu.VMEM_SHARED((8, _NS, _L), jnp.int32),
        pltpu.VMEM_SHARED((4, _NS, _L), jnp.float32),
        pltpu.SemaphoreType.DMA,
    ],
)


def kernel(t_cls_0, t_cls_1, t_cls_2, t_bbox_0, t_bbox_1, t_bbox_2,
           t_ang_0, t_ang_1, t_ang_2, t_cent_0, t_cent_1, t_cent_2,
           s_cls_0, s_cls_1, s_cls_2, s_bbox_0, s_bbox_1, s_bbox_2,
           s_ang_0, s_ang_1, s_ang_2, s_cent_0, s_cent_1, s_cent_2):
    args = (t_cls_0, t_cls_1, t_cls_2, t_bbox_0, t_bbox_1, t_bbox_2,
            t_ang_0, t_ang_1, t_ang_2, t_cent_0, t_cent_1, t_cent_2,
            s_cls_0, s_cls_1, s_cls_2, s_bbox_0, s_bbox_1, s_bbox_2,
            s_ang_0, s_ang_1, s_ang_2, s_cent_0, s_cent_1, s_cent_2)
    big0, big1, big0u, big1u, smalls = _dense_stage(args)
    out = _select_stage()(big0.reshape(-1), big1.reshape(-1),
                          big0u.reshape(-1), big1u.reshape(-1),
                          smalls.reshape(-1))
    loss_cls = (out[0] + out[_L]) * 0.5
    loss_bbox = (out[1] + out[_L + 1]) * 0.5
    loss_cent = (out[2] + out[_L + 2]) * 0.5
    return (loss_cls, loss_bbox, loss_cent)


# speedup vs baseline: 1.1094x; 1.0257x over previous
import functools

import jax
import jax.numpy as jnp
import numpy as np
from jax import lax
from jax.experimental import pallas as pl
from jax.experimental.pallas import tpu as pltpu
from jax.experimental.pallas import tpu_sc as plsc

_CLS = 16
_SIZES = [128, 64, 32]
_B = 2
_K = 2000
_NS = 16
_L = 16

_C0 = _SIZES[0] * _SIZES[0] // _NS
_C1 = _SIZES[1] * _SIZES[1] // _NS

_FINE_I = int(np.frombuffer(np.float32(0.02).tobytes(), np.int32)[0])


def _bce(p, t):
    return -(t * jnp.clip(jnp.log(p), -100.0, None)
             + (1.0 - t) * jnp.clip(jnp.log(1.0 - p), -100.0, None))


def _smooth_l1(a, b):
    d = jnp.abs(a - b)
    return jnp.where(d < 1.0, 0.5 * d * d, d - 0.5)


def _dense_body(*refs):
    ins = refs[:24]
    big0, big1, big0u, big1u, smalls = refs[24:]
    (t_cls, t_bbox, t_ang, t_cent, s_cls, s_bbox, s_ang, s_cent) = (
        ins[0:3], ins[3:6], ins[6:9], ins[9:12],
        ins[12:15], ins[15:18], ins[18:21], ins[21:24])

    for img in range(1):
        ln_total = jnp.float32(0.0)
        neg_num = jnp.float32(0.0)
        mv_sum = jnp.float32(0.0)
        npos2 = jnp.float32(0.0)
        wm2 = jnp.float32(0.0)
        dlp2 = jnp.float32(0.0)
        pb2 = jnp.float32(0.0)
        pc2 = jnp.float32(0.0)

        for lvl in range(3):
            tcs = jax.nn.sigmoid(t_cent[lvl][img, 0])
            tp = jax.nn.sigmoid(t_cls[lvl][img])
            if lvl < 2:
                mv = jnp.max(tp * tcs[None], axis=0)
            else:
                mv = jnp.max(tp, axis=0)

            sx = s_cls[lvl][img]
            z = jnp.exp(-sx)
            ssig = 1.0 / (1.0 + z)
            sp = jnp.log(1.0 + z)
            logp = jnp.clip(-sp, -100.0, None)
            log1mp = jnp.clip(-sx - sp, -100.0, None)
            lp = jnp.sum(-(tp * logp + (1.0 - tp) * log1mp)
                         * (tp - ssig) ** 2, axis=0)
            ln = jnp.sum(-log1mp * ssig ** 2, axis=0)

            bb = jnp.sum(_smooth_l1(s_bbox[lvl][img], t_bbox[lvl][img]), axis=0)
            bb = bb + _smooth_l1(s_ang[lvl][img, 0], t_ang[lvl][img, 0])
            cx = s_cent[lvl][img, 0]
            z2 = jnp.exp(-cx)
            sp2 = jnp.log(1.0 + z2)
            ce = -(tcs * jnp.clip(-sp2, -100.0, None)
                   + (1.0 - tcs) * jnp.clip(-cx - sp2, -100.0, None))

            dlp = lp - ln
            pb = mv * bb
            pc = mv * ce

            ln_total = ln_total + jnp.sum(ln)
            mv_sum = mv_sum + jnp.sum(mv)
            neg_num = neg_num + jnp.sum(jnp.where(mv > 0.0, lp, ln))

            if lvl == 0:
                big0[img, 0, :, :] = mv
                big0[img, 1, :, :] = dlp
                big0[img, 2, :, :] = pb
                big0[img, 3, :, :] = pc
                big0u[img, :, :] = lax.bitcast_convert_type(mv, jnp.int32)
            elif lvl == 1:
                big1[img, 0, :, :] = mv
                big1[img, 1, :, :] = dlp
                big1[img, 2, :, :] = pb
                big1[img, 3, :, :] = pc
                big1u[img, :, :] = lax.bitcast_convert_type(mv, jnp.int32)
            else:
                m2 = mv > 0.02
                npos2 = npos2 + jnp.sum(m2.astype(jnp.float32))
                wm2 = wm2 + jnp.sum(jnp.where(m2, mv, 0.0))
                dlp2 = dlp2 + jnp.sum(jnp.where(m2, dlp, 0.0))
                pb2 = pb2 + jnp.sum(jnp.where(m2, pb, 0.0))
                pc2 = pc2 + jnp.sum(jnp.where(m2, pc, 0.0))

        lane = lax.broadcasted_iota(jnp.int32, (1, 128), 1)
        row = jnp.zeros((1, 128), jnp.float32)
        for k, v in enumerate([ln_total, neg_num, mv_sum, npos2, wm2,
                               dlp2, pb2, pc2]):
            row = jnp.where(lane == k, v, row)
        smalls[0, :, :] = row


def _dense_stage(args):
    def ispec(c, s):
        return pl.BlockSpec((1, c, s, s), lambda i: (i, 0, 0, 0))
    in_specs = []
    for pfx in range(2):
        for c in (_CLS, 4, 1, 1):
            for s in _SIZES:
                in_specs.append(ispec(c, s))
    return pl.pallas_call(
        _dense_body,
        grid=(_B,),
        in_specs=in_specs,
        out_specs=[
            pl.BlockSpec((1, 4, 128, 128), lambda i: (i, 0, 0, 0)),
            pl.BlockSpec((1, 4, 64, 64), lambda i: (i, 0, 0, 0)),
            pl.BlockSpec((1, 128, 128), lambda i: (i, 0, 0)),
            pl.BlockSpec((1, 64, 64), lambda i: (i, 0, 0)),
            pl.BlockSpec((1, 1, 128), lambda i: (i, 0, 0)),
        ],
        out_shape=[
            jax.ShapeDtypeStruct((_B, 4, 128, 128), jnp.float32),
            jax.ShapeDtypeStruct((_B, 4, 64, 64), jnp.float32),
            jax.ShapeDtypeStruct((_B, 128, 128), jnp.int32),
            jax.ShapeDtypeStruct((_B, 64, 64), jnp.int32),
            jax.ShapeDtypeStruct((_B, 1, 128), jnp.float32),
        ],
    )(*args)


def _lane_iota():
    return lax.iota(jnp.int32, _L)


def _bcast(vec, lane):
    return vec[jnp.full((_L,), lane, jnp.int32)]


def _lanesum(vec):
    li = _lane_iota()
    for d in (1, 2, 4, 8):
        vec = vec + vec[li ^ d]
    return vec


def _prefix_incl(x):
    li = _lane_iota()
    for d in (1, 2, 4, 8):
        shifted = x[jnp.maximum(li - d, 0)]
        x = x + jnp.where(li >= d, shifted, 0)
    return x


def _count_chunks(b0i, b1i, p0, p1, strict):
    cnt0 = jnp.zeros((_L,), jnp.int32)
    for j in range(_C0 // _L):
        u = b0i[pl.ds(j * _L, _L)]
        m = (u > p0) if strict else (u >= p0)
        cnt0 = cnt0 + jnp.where(m, 1, 0)
    cnt1 = jnp.zeros((_L,), jnp.int32)
    for j in range(_C1 // _L):
        u = b1i[pl.ds(j * _L, _L)]
        m = (u > p1) if strict else (u >= p1)
        cnt1 = cnt1 + jnp.where(m, 1, 0)
    return _lanesum(cnt0), _lanesum(cnt1)


def _count3_chunks(b0i, b1i, ps0, ps1):
    c0 = [jnp.zeros((_L,), jnp.int32) for _ in range(3)]
    for j in range(_C0 // _L):
        u = b0i[pl.ds(j * _L, _L)]
        for k in range(3):
            c0[k] = c0[k] + jnp.where(u >= ps0[k], 1, 0)
    c1 = [jnp.zeros((_L,), jnp.int32) for _ in range(3)]
    for j in range(_C1 // _L):
        u = b1i[pl.ds(j * _L, _L)]
        for k in range(3):
            c1[k] = c1[k] + jnp.where(u >= ps1[k], 1, 0)
    return [_lanesum(v) for v in c0], [_lanesum(v) for v in c1]


def _select_body(big0, big1, big0u, big1u, smalls, out, buf0, buf1, b0i, b1i,
                 sbuf, pubi, rdi, pubf, rdf, outbuf, shared_i, shared_f, sem):
    c = lax.axis_index("c")
    s = lax.axis_index("s")
    li = _lane_iota()

    cps = []
    for arr in range(4):
        cps.append(pltpu.async_copy(
            big0.at[pl.ds((c * 4 + arr) * 16384 + s * _C0, _C0)],
            buf0.at[arr], sem))
        cps.append(pltpu.async_copy(
            big1.at[pl.ds((c * 4 + arr) * 4096 + s * _C1, _C1)],
            buf1.at[arr], sem))
    cps.append(pltpu.async_copy(
        big0u.at[pl.ds(c * 16384 + s * _C0, _C0)], b0i, sem))
    cps.append(pltpu.async_copy(
        big1u.at[pl.ds(c * 4096 + s * _C1, _C1)], b1i, sem))
    cps.append(pltpu.async_copy(smalls.at[pl.ds(c * 128, 128)], sbuf, sem))
    for cp in cps:
        cp.wait()

    def round_fn(i, lohi):
        lo, hi = lohi
        d = (hi - lo) >> 2
        p1 = lo + d
        p2 = lo + d + d
        p3 = lo + d + d + d
        c0s, c1s = _count3_chunks(
            b0i, b1i,
            [_bcast(p1, 0), _bcast(p2, 0), _bcast(p3, 0)],
            [_bcast(p1, 1), _bcast(p2, 1), _bcast(p3, 1)])
        pub = jnp.zeros((_L,), jnp.int32)
        for k in range(3):
            pub = (pub + jnp.where(li == k, c0s[k], 0)
                   + jnp.where(li == 3 + k, c1s[k], 0))
        pubi[...] = pub
        slot = 4 + lax.rem(i, 2)
        pltpu.sync_copy(pubi, shared_i.at[slot, s])
        plsc.subcore_barrier()
        pltpu.sync_copy(shared_i.at[slot], rdi)
        tot = jnp.zeros((_L,), jnp.int32)
        for r in range(_NS):
            tot = tot + rdi[r, :]
        sel = jnp.minimum(li, 1) * 3
        g1 = tot[sel] >= _K
        g2 = tot[sel + 1] >= _K
        g3 = tot[sel + 2] >= _K
        lo2 = jnp.where(g3, p3, jnp.where(g2, p2, jnp.where(g1, p1, lo)))
        hi2 = jnp.where(g3, hi, jnp.where(g2, p3, jnp.where(g1, p2, p1)))
        return (lo2, hi2)

    lo0 = jnp.zeros((_L,), jnp.int32)
    hi0 = jnp.full((_L,), 1 << 30, jnp.int32)
    lo, _ = lax.fori_loop(0, 15, round_fn, (lo0, hi0))
    t0 = _bcast(lo, 0)
    t1 = _bcast(lo, 1)

    g0, g1 = _count_chunks(b0i, b1i, t0, t1, strict=True)
    e0 = jnp.zeros((_L,), jnp.int32)
    for j in range(_C0 // _L):
        u = b0i[pl.ds(j * _L, _L)]
        e0 = e0 + jnp.where(u == t0, 1, 0)
    e1 = jnp.zeros((_L,), jnp.int32)
    for j in range(_C1 // _L):
        u = b1i[pl.ds(j * _L, _L)]
        e1 = e1 + jnp.where(u == t1, 1, 0)
    pub = (jnp.where(li == 0, g0, 0) + jnp.where(li == 1, g1, 0)
           + jnp.where(li == 2, _lanesum(e0), 0)
           + jnp.where(li == 3, _lanesum(e1), 0))
    pubi[...] = pub
    pltpu.sync_copy(pubi, shared_i.at[6, s])
    plsc.subcore_barrier()
    pltpu.sync_copy(shared_i.at[6], rdi)
    s_vec = jnp.zeros((_L,), jnp.int32) + s
    tot = jnp.zeros((_L,), jnp.int32)
    pref = jnp.zeros((_L,), jnp.int32)
    for r in range(_NS):
        row = rdi[r, :]
        tot = tot + row
        pref = pref + row * jnp.minimum(jnp.maximum(s_vec - r, 0), 1)
    b0 = (_K - _bcast(tot, 0)) - _bcast(pref, 2)
    b1 = (_K - _bcast(tot, 1)) - _bcast(pref, 3)

    npos = jnp.zeros((_L,), jnp.int32)
    wm = jnp.zeros((_L,), jnp.float32)
    dlp = jnp.zeros((_L,), jnp.float32)
    pb = jnp.zeros((_L,), jnp.float32)
    pc = jnp.zeros((_L,), jnp.float32)
    for lvl, buf, bi, n, tt, bb in ((0, buf0, b0i, _C0, t0, b0),
                                    (1, buf1, b1i, _C1, t1, b1)):
        carry = jnp.zeros((_L,), jnp.int32)
        for j in range(n // _L):
            sl = pl.ds(j * _L, _L)
            mvv = buf[0, sl]
            u = bi[sl]
            tie_i = jnp.where(u == tt, 1, 0)
            incl = _prefix_incl(tie_i)
            cond_i = jnp.minimum(jnp.maximum(bb - incl - carry + 1, 0), 1)
            gt_i = jnp.where(u > tt, 1, 0)
            sel_i = gt_i + tie_i * cond_i
            carry = carry + _bcast(incl, _L - 1)
            fine_i = jnp.where(u > _FINE_I, 1, 0)
            m_i = sel_i * fine_i
            m_f = m_i.astype(jnp.float32)
            npos = npos + m_i
            wm = wm + mvv * m_f
            dlp = dlp + buf[1, sl] * m_f
            pb = pb + buf[2, sl] * m_f
            pc = pc + buf[3, sl] * m_f

    pubv = (jnp.where(li == 0, _lanesum(npos).astype(jnp.float32), 0.0)
            + jnp.where(li == 1, _lanesum(wm), 0.0)
            + jnp.where(li == 2, _lanesum(dlp), 0.0)
            + jnp.where(li == 3, _lanesum(pb), 0.0)
            + jnp.where(li == 4, _lanesum(pc), 0.0))
    pubf[...] = pubv
    pltpu.sync_copy(pubf, shared_f.at[2, s])
    plsc.subcore_barrier()

    @pl.when(s == 0)
    def _():
        pltpu.sync_copy(shared_f.at[2], rdf)
        acc = jnp.zeros((_L,), jnp.float32)
        for r in range(_NS):
            acc = acc + rdf[r, :]
        sm = sbuf[pl.ds(0, _L)]
        ln_total = _bcast(sm, 0)
        neg_num = _bcast(sm, 1)
        mv_sum = _bcast(sm, 2)
        npos_f = _bcast(acc, 0) + _bcast(sm, 3)
        wm_sum = _bcast(acc, 1) + _bcast(sm, 4)
        dlp_sum = _bcast(acc, 2) + _bcast(sm, 5)
        pb_sum = _bcast(acc, 3) + _bcast(sm, 6)
        pc_sum = _bcast(acc, 4) + _bcast(sm, 7)

        hp = jnp.minimum(npos_f, 1.0)
        pos_num = ln_total + dlp_sum
        wm_safe = wm_sum * hp + (1.0 - hp)
        loss_cls = hp * (pos_num / wm_safe) + (1.0 - hp) * (neg_num / mv_sum)
        npf = jnp.maximum(npos_f, 1.0)
        loss_bbox = hp * (pb_sum / (npf * 5.0) * 10.0)
        loss_cent = hp * (pc_sum / npf * 10.0)

        ov = (jnp.where(li == 0, loss_cls, 0.0)
              + jnp.where(li == 1, loss_bbox, 0.0)
              + jnp.where(li == 2, loss_cent, 0.0))
        outbuf[...] = ov
        pltpu.sync_copy(outbuf, out.at[pl.ds(c * _L, _L)])


_select_stage = functools.partial(
    pl.kernel,
    _select_body,
    out_type=jax.ShapeDtypeStruct((2 * _L,), jnp.float32),
    mesh=plsc.VectorSubcoreMesh(core_axis_name="c", subcore_axis_name="s",
                                num_cores=2, num_subcores=_NS),
    scratch_types=[
        pltpu.VMEM((4, _C0), jnp.float32),
        pltpu.VMEM((4, _C1), jnp.float32),
        pltpu.VMEM((_C0,), jnp.int32),
        pltpu.VMEM((_C1,), jnp.int32),
        pltpu.VMEM((128,), jnp.float32),
        pltpu.VMEM((_L,), jnp.int32),
        pltpu.VMEM((_NS, _L), jnp.int32),
        pltpu.VMEM((_L,), jnp.float32),
        pltpu.VMEM((_NS, _L), jnp.float32),
        pltpu.VMEM((_L,), jnp.float32),
        pltpu.VMEM_SHARED((8, _NS, _L), jnp.int32),
        pltpu.VMEM_SHARED((4, _NS, _L), jnp.float32),
        pltpu.SemaphoreType.DMA,
    ],
)


def kernel(t_cls_0, t_cls_1, t_cls_2, t_bbox_0, t_bbox_1, t_bbox_2,
           t_ang_0, t_ang_1, t_ang_2, t_cent_0, t_cent_1, t_cent_2,
           s_cls_0, s_cls_1, s_cls_2, s_bbox_0, s_bbox_1, s_bbox_2,
           s_ang_0, s_ang_1, s_ang_2, s_cent_0, s_cent_1, s_cent_2):
    args = (t_cls_0, t_cls_1, t_cls_2, t_bbox_0, t_bbox_1, t_bbox_2,
            t_ang_0, t_ang_1, t_ang_2, t_cent_0, t_cent_1, t_cent_2,
            s_cls_0, s_cls_1, s_cls_2, s_bbox_0, s_bbox_1, s_bbox_2,
            s_ang_0, s_ang_1, s_ang_2, s_cent_0, s_cent_1, s_cent_2)
    big0, big1, big0u, big1u, smalls = _dense_stage(args)
    out = _select_stage()(big0.reshape(-1), big1.reshape(-1),
                          big0u.reshape(-1), big1u.reshape(-1),
                          smalls.reshape(-1))
    loss_cls = (out[0] + out[_L]) * 0.5
    loss_bbox = (out[1] + out[_L + 1]) * 0.5
    loss_cent = (out[2] + out[_L + 2]) * 0.5
    return (loss_cls, loss_bbox, loss_cent)
